# baseline (speedup 1.0000x reference)
import functools

import jax
import jax.numpy as jnp
from jax import lax
from jax.experimental import pallas as pl
from jax.experimental.pallas import tpu as pltpu
from jax.experimental.pallas import tpu_sc as plsc

N = 50000
E = 800000
H = 256
LAYERS = 10

NP = 50176
E2 = E + N
E2P = 851968
ER = E2P // 128
CH = 16
CW = 16
RPT = NP // 16
BLK = 98
TB = 512

_mesh = plsc.VectorSubcoreMesh(core_axis_name="c", subcore_axis_name="s")


@functools.partial(
    pl.kernel,
    out_type=jax.ShapeDtypeStruct((2 * NP,), jnp.float32),
    mesh=_mesh,
    scratch_types=[
        pltpu.VMEM((1024,), jnp.int32),
        pltpu.VMEM((1024,), jnp.float32),
        pltpu.VMEM((RPT,), jnp.float32),
        pltpu.VMEM_SHARED((NP,), jnp.float32),
    ],
)
def _deg_sc(dst_hbm, ones_hbm, zeros_hbm, degp_hbm, didx, ones, zbuf, acc):
    c = lax.axis_index("c")
    s = lax.axis_index("s")
    pltpu.sync_copy(ones_hbm, ones)
    pltpu.sync_copy(zeros_hbm, zbuf)
    pltpu.sync_copy(zbuf, acc.at[pl.ds(s * RPT, RPT)])
    plsc.subcore_barrier()
    w = c * 16 + s

    def blk(b, carry):
        pltpu.sync_copy(dst_hbm.at[pl.ds(w * 26624 + b * 1024, 1024)], didx)
        pltpu.sync_copy(ones, acc.at[didx], add=True)
        return carry

    lax.fori_loop(0, 26, blk, 0)
    plsc.subcore_barrier()
    pltpu.sync_copy(acc.at[pl.ds(s * RPT, RPT)], zbuf)
    pltpu.sync_copy(zbuf, degp_hbm.at[pl.ds(c * NP + s * RPT, RPT)])


@functools.partial(
    pl.kernel,
    out_type=jax.ShapeDtypeStruct((2 * NP,), jnp.float32),
    mesh=_mesh,
    scratch_types=[
        pltpu.VMEM((1024,), jnp.int32),
        pltpu.VMEM((1024,), jnp.int32),
        pltpu.VMEM((1024,), jnp.float32),
        pltpu.VMEM((RPT,), jnp.float32),
        pltpu.VMEM_SHARED((NP,), jnp.float32),
        pltpu.SemaphoreType.DMA,
    ],
)
def _seg1_sc(xs_hbm, src_hbm, dst_hbm, zeros_hbm, outp_hbm,
             sidx, didx, vals, zbuf, acc, sem):
    c = lax.axis_index("c")
    s = lax.axis_index("s")
    pltpu.sync_copy(zeros_hbm, zbuf)
    pltpu.sync_copy(zbuf, acc.at[pl.ds(s * RPT, RPT)])
    plsc.subcore_barrier()
    w = c * 16 + s

    def blk(b, carry):
        r0 = w * 26624 + b * 1024
        pltpu.sync_copy(src_hbm.at[pl.ds(r0, 1024)], sidx)
        pltpu.sync_copy(dst_hbm.at[pl.ds(r0, 1024)], didx)
        pltpu.async_copy(xs_hbm.at[sidx], vals, sem).wait()
        pltpu.sync_copy(vals, acc.at[didx], add=True)
        return carry

    lax.fori_loop(0, 26, blk, 0)
    plsc.subcore_barrier()
    pltpu.sync_copy(acc.at[pl.ds(s * RPT, RPT)], zbuf)
    pltpu.sync_copy(zbuf, outp_hbm.at[pl.ds(c * NP + s * RPT, RPT)])


@functools.partial(
    pl.kernel,
    out_type=jax.ShapeDtypeStruct((NP, CH, CW), jnp.float32),
    mesh=_mesh,
    scratch_types=[
        [pltpu.VMEM((512,), jnp.int32) for _ in range(4)],
        [pltpu.VMEM((512,), jnp.int32) for _ in range(4)],
        [pltpu.VMEM((512, CW), jnp.float32) for _ in range(4)],
        pltpu.VMEM((784, CW), jnp.float32),
        pltpu.VMEM_SHARED((NP, CW), jnp.float32),
        [pltpu.SemaphoreType.DMA for _ in range(4)],
        [pltpu.SemaphoreType.DMA for _ in range(4)],
    ],
    compiler_params=pltpu.CompilerParams(use_tc_tiling_on_sc=False),
)
def _seg_sc(h0, h1, h2, h3, h4, h5, h6, h7, h8, h9, h10, h11, h12, h13,
            h14, h15, src_hbm, dst_hbm, zeros_hbm, t_hbm,
            sidx, didx, gbuf, zbuf, acc, gsem, ssem):
    c = lax.axis_index("c")
    s = lax.axis_index("s")
    tables = [h0, h1, h2, h3, h4, h5, h6, h7,
              h8, h9, h10, h11, h12, h13, h14, h15]
    pltpu.sync_copy(zeros_hbm, zbuf)
    for chunk in range(CH):
        table = tables[chunk]
        mine = c == (chunk // (CH // 2))
        e0 = s * 53248

        def _idx_gather(b, j):
            pltpu.sync_copy(src_hbm.at[pl.ds(e0 + b * 512, 512)], sidx[j])
            pltpu.sync_copy(dst_hbm.at[pl.ds(e0 + b * 512, 512)], didx[j])
            pltpu.async_copy(table.at[sidx[j]], gbuf[j], gsem[j])

        def _wait_scatter(j):
            pltpu.make_async_copy(gbuf[j], acc.at[didx[j]], ssem[j]).wait()

        def _scatter(j):
            pltpu.make_async_copy(table.at[sidx[j]], gbuf[j], gsem[j]).wait()
            pltpu.async_copy(gbuf[j], acc.at[didx[j]], ssem[j], add=True)

        @pl.when(mine)
        def _zero():
            for z in range(4):
                pltpu.sync_copy(zbuf, acc.at[pl.ds(s * RPT + z * 784, 784)])

        plsc.subcore_barrier()

        @pl.when(mine)
        def _edges():
            def grp(i, carry):
                @pl.when(i > 0)
                def _drain():
                    for j in range(4):
                        _wait_scatter(j)

                for j in range(4):
                    _idx_gather(i * 4 + j, j)
                for j in range(4):
                    _scatter(j)
                return carry

            lax.fori_loop(0, 26, grp, 0)
            for j in range(4):
                _wait_scatter(j)

        plsc.subcore_barrier()

        @pl.when(mine)
        def _out():
            for z in range(4):
                pltpu.sync_copy(acc.at[pl.ds(s * RPT + z * 784, 784)],
                                gbuf[0].at[pl.ds(0, 784)])
                pltpu.sync_copy(gbuf[0].at[pl.ds(0, 784)],
                                t_hbm.at[pl.ds(s * RPT + z * 784, 784), chunk])

        plsc.subcore_barrier()


def _prep_body(x_ref, da_ref, db_ref, dinv_ref, xs_ref):
    deg = da_ref[...] + db_ref[...]
    dinv = lax.rsqrt(jnp.maximum(deg, 1e-12))
    dinv_ref[...] = dinv[:, None]
    xs_ref[...] = dinv * x_ref[:, 0]


def _prep_tc(xpad, degp):
    return pl.pallas_call(
        _prep_body,
        grid=(BLK,),
        in_specs=[
            pl.BlockSpec((TB, 1), lambda i: (i, 0)),
            pl.BlockSpec((TB,), lambda i: (i,)),
            pl.BlockSpec((TB,), lambda i: (i + BLK,)),
        ],
        out_specs=[
            pl.BlockSpec((TB, 1), lambda i: (i, 0)),
            pl.BlockSpec((TB,), lambda i: (i,)),
        ],
        out_shape=[
            jax.ShapeDtypeStruct((NP, 1), jnp.float32),
            jax.ShapeDtypeStruct((NP,), jnp.float32),
        ],
    )(xpad, degp, degp)


def _l1_body(ta_ref, tb_ref, dinv_ref, w1_ref, b1_ref, *out_refs):
    t1 = ta_ref[...] + tb_ref[...]
    dinv = dinv_ref[:, 0]
    u = dinv * t1
    y = jax.nn.relu(u[:, None] * w1_ref[0, :][None, :] + b1_ref[...])
    hs = dinv[:, None] * y
    for cc in range(CH):
        out_refs[cc][...] = hs[:, cc * CW:(cc + 1) * CW]


def _l1_tc(t1p, dinv, W1, b1r):
    return pl.pallas_call(
        _l1_body,
        grid=(BLK,),
        in_specs=[
            pl.BlockSpec((TB,), lambda i: (i,)),
            pl.BlockSpec((TB,), lambda i: (i + BLK,)),
            pl.BlockSpec((TB, 1), lambda i: (i, 0)),
            pl.BlockSpec((1, H), lambda i: (0, 0)),
            pl.BlockSpec((1, H), lambda i: (0, 0)),
        ],
        out_specs=[pl.BlockSpec((TB, CW), lambda i: (i, 0))
                   for _ in range(CH)],
        out_shape=[jax.ShapeDtypeStruct((NP, CW), jnp.float32)
                   for _ in range(CH)],
    )(t1p, t1p, dinv, W1, b1r)


def _mid_body(t_ref, dinv_ref, w_ref, b_ref, *out_refs):
    dinv = dinv_ref[...]
    u = dinv * t_ref[...].reshape(TB, H)
    y = jax.nn.relu(
        jnp.dot(u, w_ref[...], preferred_element_type=jnp.float32)
        + b_ref[...])
    hs = dinv * y
    for cc in range(CH):
        out_refs[cc][...] = hs[:, cc * CW:(cc + 1) * CW]


def _mid_tc(t, dinv, W, br):
    return pl.pallas_call(
        _mid_body,
        grid=(BLK,),
        in_specs=[
            pl.BlockSpec((TB * H,), lambda i: (i,)),
            pl.BlockSpec((TB, 1), lambda i: (i, 0)),
            pl.BlockSpec((H, H), lambda i: (0, 0)),
            pl.BlockSpec((1, H), lambda i: (0, 0)),
        ],
        out_specs=[pl.BlockSpec((TB, CW), lambda i: (i, 0))
                   for _ in range(CH)],
        out_shape=[jax.ShapeDtypeStruct((NP, CW), jnp.float32)
                   for _ in range(CH)],
    )(t, dinv, W, br)


def _end_body(t_ref, dinv_ref, w_ref, b_ref, wc_ref, bc_ref, out_ref):
    dinv = dinv_ref[...]
    u = dinv * t_ref[...].reshape(TB, H)
    y = jax.nn.relu(
        jnp.dot(u, w_ref[...], preferred_element_type=jnp.float32)
        + b_ref[...])
    out_ref[...] = (
        jnp.dot(y, wc_ref[...], preferred_element_type=jnp.float32)
        + bc_ref[...])


def _end_tc(t, dinv, W, br, Wc, bcr):
    return pl.pallas_call(
        _end_body,
        grid=(BLK,),
        in_specs=[
            pl.BlockSpec((TB * H,), lambda i: (i,)),
            pl.BlockSpec((TB, 1), lambda i: (i, 0)),
            pl.BlockSpec((H, H), lambda i: (0, 0)),
            pl.BlockSpec((1, H), lambda i: (0, 0)),
            pl.BlockSpec((H, 1), lambda i: (0, 0)),
            pl.BlockSpec((1, 1), lambda i: (0, 0)),
        ],
        out_specs=pl.BlockSpec((TB, 1), lambda i: (i, 0)),
        out_shape=jax.ShapeDtypeStruct((NP, 1), jnp.float32),
    )(t, dinv, W, br, Wc, bcr)


def kernel(x, edge_index, W1, b1, W, b, Wc, bc):
    loops = jnp.arange(N, dtype=jnp.int32)
    pad_e = E2P - E2
    src2 = jnp.concatenate(
        [edge_index[0], loops, jnp.zeros((pad_e,), jnp.int32)])
    dst2 = jnp.concatenate(
        [edge_index[1], loops, jnp.full((pad_e,), NP - 1, jnp.int32)])
    dstr = dst2

    xpad = jnp.zeros((NP, 1), jnp.float32).at[:N].set(x)
    ones2d = jnp.ones((1024,), jnp.float32)
    zeros1 = jnp.zeros((RPT,), jnp.float32)
    zeros32 = jnp.zeros((784, CW), jnp.float32)
    b1r = b1.reshape(1, H)
    br = b.reshape(1, H)
    bcr = bc.reshape(1, 1)

    degp = _deg_sc(dstr, ones2d, zeros1)
    dinv, xs = _prep_tc(xpad, degp)
    t1p = _seg1_sc(xs, src2, dstr, zeros1)
    hsf = _l1_tc(t1p, dinv, W1, b1r)
    for _ in range(2, LAYERS):
        t = _seg_sc(*hsf, src2, dstr, zeros32)
        hsf = _mid_tc(t.reshape(NP * H), dinv, W, br)
    t = _seg_sc(*hsf, src2, dstr, zeros32)
    out = _end_tc(t.reshape(NP * H), dinv, W, br, Wc, bcr)
    return out[:N]


# speedup vs baseline: 1.3147x; 1.3147x over previous
import functools

import jax
import jax.numpy as jnp
from jax import lax
from jax.experimental import pallas as pl
from jax.experimental.pallas import tpu as pltpu
from jax.experimental.pallas import tpu_sc as plsc

N = 50000
E = 800000
H = 256
LAYERS = 10

NP = 50176
E2 = E + N
E2P = 851968
ER = E2P // 128
CH = 16
CW = 16
RPT = NP // 16
BLK = 98
TB = 512

_mesh = plsc.VectorSubcoreMesh(core_axis_name="c", subcore_axis_name="s")


@functools.partial(
    pl.kernel,
    out_type=jax.ShapeDtypeStruct((2 * NP,), jnp.float32),
    mesh=_mesh,
    scratch_types=[
        pltpu.VMEM((1024,), jnp.int32),
        pltpu.VMEM((1024,), jnp.float32),
        pltpu.VMEM((RPT,), jnp.float32),
        pltpu.VMEM_SHARED((NP,), jnp.float32),
    ],
)
def _deg_sc(dst_hbm, ones_hbm, zeros_hbm, degp_hbm, didx, ones, zbuf, acc):
    c = lax.axis_index("c")
    s = lax.axis_index("s")
    pltpu.sync_copy(ones_hbm, ones)
    pltpu.sync_copy(zeros_hbm, zbuf)
    pltpu.sync_copy(zbuf, acc.at[pl.ds(s * RPT, RPT)])
    plsc.subcore_barrier()
    w = c * 16 + s

    def blk(b, carry):
        pltpu.sync_copy(dst_hbm.at[pl.ds(w * 26624 + b * 1024, 1024)], didx)
        pltpu.sync_copy(ones, acc.at[didx], add=True)
        return carry

    lax.fori_loop(0, 26, blk, 0)
    plsc.subcore_barrier()
    pltpu.sync_copy(acc.at[pl.ds(s * RPT, RPT)], zbuf)
    pltpu.sync_copy(zbuf, degp_hbm.at[pl.ds(c * NP + s * RPT, RPT)])


@functools.partial(
    pl.kernel,
    out_type=jax.ShapeDtypeStruct((2 * NP,), jnp.float32),
    mesh=_mesh,
    scratch_types=[
        pltpu.VMEM((1024,), jnp.int32),
        pltpu.VMEM((1024,), jnp.int32),
        pltpu.VMEM((1024,), jnp.float32),
        pltpu.VMEM((RPT,), jnp.float32),
        pltpu.VMEM_SHARED((NP,), jnp.float32),
        pltpu.SemaphoreType.DMA,
    ],
)
def _seg1_sc(xs_hbm, src_hbm, dst_hbm, zeros_hbm, outp_hbm,
             sidx, didx, vals, zbuf, acc, sem):
    c = lax.axis_index("c")
    s = lax.axis_index("s")
    pltpu.sync_copy(zeros_hbm, zbuf)
    pltpu.sync_copy(zbuf, acc.at[pl.ds(s * RPT, RPT)])
    plsc.subcore_barrier()
    w = c * 16 + s

    def blk(b, carry):
        r0 = w * 26624 + b * 1024
        pltpu.sync_copy(src_hbm.at[pl.ds(r0, 1024)], sidx)
        pltpu.sync_copy(dst_hbm.at[pl.ds(r0, 1024)], didx)
        pltpu.async_copy(xs_hbm.at[sidx], vals, sem).wait()
        pltpu.sync_copy(vals, acc.at[didx], add=True)
        return carry

    lax.fori_loop(0, 26, blk, 0)
    plsc.subcore_barrier()
    pltpu.sync_copy(acc.at[pl.ds(s * RPT, RPT)], zbuf)
    pltpu.sync_copy(zbuf, outp_hbm.at[pl.ds(c * NP + s * RPT, RPT)])


EB = 1664


@functools.partial(
    pl.kernel,
    out_type=jax.ShapeDtypeStruct((NP, CH, CW), jnp.float32),
    mesh=_mesh,
    scratch_types=[
        [pltpu.VMEM((2, EB), jnp.int32) for _ in range(2)],
        [pltpu.VMEM((EB, CW), jnp.float32) for _ in range(2)],
        pltpu.VMEM_SHARED((NP, CW), jnp.float32),
        [pltpu.SemaphoreType.DMA for _ in range(2)],
        [pltpu.SemaphoreType.DMA for _ in range(2)],
    ],
    compiler_params=pltpu.CompilerParams(use_tc_tiling_on_sc=False),
)
def _seg_sc(h0, h1, h2, h3, h4, h5, h6, h7, h8, h9, h10, h11, h12, h13,
            h14, h15, sd_hbm, zeros_hbm, t_hbm, sd, gbuf, acc, gsem, ssem):
    c = lax.axis_index("c")
    s = lax.axis_index("s")
    tables = [h0, h1, h2, h3, h4, h5, h6, h7,
              h8, h9, h10, h11, h12, h13, h14, h15]
    def _run(chunk, table, mine):
        e0 = s * 53248

        def _idx_gather(b, j):
            pltpu.sync_copy(sd_hbm.at[:, pl.ds(e0 + b * EB, EB)], sd[j])
            pltpu.async_copy(table.at[sd[j].at[0]], gbuf[j], gsem[j])

        def _wait_scatter(j):
            pltpu.make_async_copy(gbuf[j], acc.at[sd[j].at[1]],
                                  ssem[j]).wait()

        def _scatter(j):
            pltpu.make_async_copy(table.at[sd[j].at[0]], gbuf[j],
                                  gsem[j]).wait()
            pltpu.async_copy(gbuf[j], acc.at[sd[j].at[1]], ssem[j], add=True)

        @pl.when(mine)
        def _zero():
            pltpu.sync_copy(zeros_hbm, gbuf[0].at[pl.ds(0, 784)])
            for z in range(4):
                pltpu.sync_copy(gbuf[0].at[pl.ds(0, 784)],
                                acc.at[pl.ds(s * RPT + z * 784, 784)])

        plsc.subcore_barrier()

        @pl.when(mine)
        def _edges():
            def grp(i, carry):
                @pl.when(i > 0)
                def _drain():
                    for j in range(2):
                        _wait_scatter(j)

                for j in range(2):
                    _idx_gather(i * 2 + j, j)
                for j in range(2):
                    _scatter(j)
                return carry

            lax.fori_loop(0, 16, grp, 0)
            for j in range(2):
                _wait_scatter(j)

        plsc.subcore_barrier()

        @pl.when(mine)
        def _out():
            for z in range(4):
                pltpu.sync_copy(acc.at[pl.ds(s * RPT + z * 784, 784)],
                                gbuf[0].at[pl.ds(0, 784)])
                pltpu.sync_copy(gbuf[0].at[pl.ds(0, 784)],
                                t_hbm.at[pl.ds(s * RPT + z * 784, 784), chunk])

        plsc.subcore_barrier()

    for chunk in range(CH):
        _run(chunk, tables[chunk], c == (chunk // (CH // 2)))


def _prep_body(x_ref, da_ref, db_ref, dinv_ref, xs_ref):
    deg = da_ref[...] + db_ref[...]
    dinv = lax.rsqrt(jnp.maximum(deg, 1e-12))
    dinv_ref[...] = dinv[:, None]
    xs_ref[...] = dinv * x_ref[:, 0]


def _prep_tc(xpad, degp):
    return pl.pallas_call(
        _prep_body,
        grid=(BLK,),
        in_specs=[
            pl.BlockSpec((TB, 1), lambda i: (i, 0)),
            pl.BlockSpec((TB,), lambda i: (i,)),
            pl.BlockSpec((TB,), lambda i: (i + BLK,)),
        ],
        out_specs=[
            pl.BlockSpec((TB, 1), lambda i: (i, 0)),
            pl.BlockSpec((TB,), lambda i: (i,)),
        ],
        out_shape=[
            jax.ShapeDtypeStruct((NP, 1), jnp.float32),
            jax.ShapeDtypeStruct((NP,), jnp.float32),
        ],
    )(xpad, degp, degp)


def _l1_body(ta_ref, tb_ref, dinv_ref, w1_ref, b1_ref, *out_refs):
    t1 = ta_ref[...] + tb_ref[...]
    dinv = dinv_ref[:, 0]
    u = dinv * t1
    y = jax.nn.relu(u[:, None] * w1_ref[0, :][None, :] + b1_ref[...])
    hs = dinv[:, None] * y
    for cc in range(CH):
        out_refs[cc][...] = hs[:, cc * CW:(cc + 1) * CW]


def _l1_tc(t1p, dinv, W1, b1r):
    return pl.pallas_call(
        _l1_body,
        grid=(BLK,),
        in_specs=[
            pl.BlockSpec((TB,), lambda i: (i,)),
            pl.BlockSpec((TB,), lambda i: (i + BLK,)),
            pl.BlockSpec((TB, 1), lambda i: (i, 0)),
            pl.BlockSpec((1, H), lambda i: (0, 0)),
            pl.BlockSpec((1, H), lambda i: (0, 0)),
        ],
        out_specs=[pl.BlockSpec((TB, CW), lambda i: (i, 0))
                   for _ in range(CH)],
        out_shape=[jax.ShapeDtypeStruct((NP, CW), jnp.float32)
                   for _ in range(CH)],
    )(t1p, t1p, dinv, W1, b1r)


def _mid_body(t_ref, dinv_ref, w_ref, b_ref, *out_refs):
    dinv = dinv_ref[...]
    u = dinv * t_ref[...].reshape(TB, H)
    y = jax.nn.relu(
        jnp.dot(u, w_ref[...], preferred_element_type=jnp.float32)
        + b_ref[...])
    hs = dinv * y
    for cc in range(CH):
        out_refs[cc][...] = hs[:, cc * CW:(cc + 1) * CW]


def _mid_tc(t, dinv, W, br):
    return pl.pallas_call(
        _mid_body,
        grid=(BLK,),
        in_specs=[
            pl.BlockSpec((TB * H,), lambda i: (i,)),
            pl.BlockSpec((TB, 1), lambda i: (i, 0)),
            pl.BlockSpec((H, H), lambda i: (0, 0)),
            pl.BlockSpec((1, H), lambda i: (0, 0)),
        ],
        out_specs=[pl.BlockSpec((TB, CW), lambda i: (i, 0))
                   for _ in range(CH)],
        out_shape=[jax.ShapeDtypeStruct((NP, CW), jnp.float32)
                   for _ in range(CH)],
    )(t, dinv, W, br)


def _end_body(t_ref, dinv_ref, w_ref, b_ref, wc_ref, bc_ref, out_ref):
    dinv = dinv_ref[...]
    u = dinv * t_ref[...].reshape(TB, H)
    y = jax.nn.relu(
        jnp.dot(u, w_ref[...], preferred_element_type=jnp.float32)
        + b_ref[...])
    out_ref[...] = (
        jnp.dot(y, wc_ref[...], preferred_element_type=jnp.float32)
        + bc_ref[...])


def _end_tc(t, dinv, W, br, Wc, bcr):
    return pl.pallas_call(
        _end_body,
        grid=(BLK,),
        in_specs=[
            pl.BlockSpec((TB * H,), lambda i: (i,)),
            pl.BlockSpec((TB, 1), lambda i: (i, 0)),
            pl.BlockSpec((H, H), lambda i: (0, 0)),
            pl.BlockSpec((1, H), lambda i: (0, 0)),
            pl.BlockSpec((H, 1), lambda i: (0, 0)),
            pl.BlockSpec((1, 1), lambda i: (0, 0)),
        ],
        out_specs=pl.BlockSpec((TB, 1), lambda i: (i, 0)),
        out_shape=jax.ShapeDtypeStruct((NP, 1), jnp.float32),
    )(t, dinv, W, br, Wc, bcr)


def kernel(x, edge_index, W1, b1, W, b, Wc, bc):
    loops = jnp.arange(N, dtype=jnp.int32)
    pad_e = E2P - E2
    src2 = jnp.concatenate(
        [edge_index[0], loops, jnp.zeros((pad_e,), jnp.int32)])
    dst2 = jnp.concatenate(
        [edge_index[1], loops, jnp.full((pad_e,), NP - 1, jnp.int32)])
    dstr = dst2

    sd2 = jnp.stack([src2, dst2])
    xpad = jnp.zeros((NP, 1), jnp.float32).at[:N].set(x)
    ones2d = jnp.ones((1024,), jnp.float32)
    zeros1 = jnp.zeros((RPT,), jnp.float32)
    zeros32 = jnp.zeros((784, CW), jnp.float32)
    b1r = b1.reshape(1, H)
    br = b.reshape(1, H)
    bcr = bc.reshape(1, 1)

    degp = _deg_sc(dstr, ones2d, zeros1)
    dinv, xs = _prep_tc(xpad, degp)
    t1p = _seg1_sc(xs, src2, dstr, zeros1)
    hsf = _l1_tc(t1p, dinv, W1, b1r)
    for _ in range(2, LAYERS):
        t = _seg_sc(*hsf, sd2, zeros32)
        hsf = _mid_tc(t.reshape(NP * H), dinv, W, br)
    t = _seg_sc(*hsf, sd2, zeros32)
    out = _end_tc(t.reshape(NP * H), dinv, W, br, Wc, bcr)
    return out[:N]


# speedup vs baseline: 1.3502x; 1.0270x over previous
import functools

import jax
import jax.numpy as jnp
from jax import lax
from jax.experimental import pallas as pl
from jax.experimental.pallas import tpu as pltpu
from jax.experimental.pallas import tpu_sc as plsc

N = 50000
E = 800000
H = 256
LAYERS = 10

NP = 50176
E2 = E + N
E2P = 851968
ER = E2P // 128
CH = 16
CW = 16
RPT = NP // 16
BLK = 49
TB = 1024

_mesh = plsc.VectorSubcoreMesh(core_axis_name="c", subcore_axis_name="s")


@functools.partial(
    pl.kernel,
    out_type=jax.ShapeDtypeStruct((2 * NP,), jnp.float32),
    mesh=_mesh,
    scratch_types=[
        pltpu.VMEM((1024,), jnp.int32),
        pltpu.VMEM((1024,), jnp.float32),
        pltpu.VMEM((RPT,), jnp.float32),
        pltpu.VMEM_SHARED((NP,), jnp.float32),
    ],
)
def _deg_sc(dst_hbm, ones_hbm, zeros_hbm, degp_hbm, didx, ones, zbuf, acc):
    c = lax.axis_index("c")
    s = lax.axis_index("s")
    pltpu.sync_copy(ones_hbm, ones)
    pltpu.sync_copy(zeros_hbm, zbuf)
    pltpu.sync_copy(zbuf, acc.at[pl.ds(s * RPT, RPT)])
    plsc.subcore_barrier()
    w = c * 16 + s

    def blk(b, carry):
        pltpu.sync_copy(dst_hbm.at[pl.ds(w * 26624 + b * 1024, 1024)], didx)
        pltpu.sync_copy(ones, acc.at[didx], add=True)
        return carry

    lax.fori_loop(0, 26, blk, 0)
    plsc.subcore_barrier()
    pltpu.sync_copy(acc.at[pl.ds(s * RPT, RPT)], zbuf)
    pltpu.sync_copy(zbuf, degp_hbm.at[pl.ds(c * NP + s * RPT, RPT)])


@functools.partial(
    pl.kernel,
    out_type=jax.ShapeDtypeStruct((2 * NP,), jnp.float32),
    mesh=_mesh,
    scratch_types=[
        pltpu.VMEM((1024,), jnp.int32),
        pltpu.VMEM((1024,), jnp.int32),
        pltpu.VMEM((1024,), jnp.float32),
        pltpu.VMEM((RPT,), jnp.float32),
        pltpu.VMEM_SHARED((NP,), jnp.float32),
        pltpu.SemaphoreType.DMA,
    ],
)
def _seg1_sc(xs_hbm, src_hbm, dst_hbm, zeros_hbm, outp_hbm,
             sidx, didx, vals, zbuf, acc, sem):
    c = lax.axis_index("c")
    s = lax.axis_index("s")
    pltpu.sync_copy(zeros_hbm, zbuf)
    pltpu.sync_copy(zbuf, acc.at[pl.ds(s * RPT, RPT)])
    plsc.subcore_barrier()
    w = c * 16 + s

    def blk(b, carry):
        r0 = w * 26624 + b * 1024
        pltpu.sync_copy(src_hbm.at[pl.ds(r0, 1024)], sidx)
        pltpu.sync_copy(dst_hbm.at[pl.ds(r0, 1024)], didx)
        pltpu.async_copy(xs_hbm.at[sidx], vals, sem).wait()
        pltpu.sync_copy(vals, acc.at[didx], add=True)
        return carry

    lax.fori_loop(0, 26, blk, 0)
    plsc.subcore_barrier()
    pltpu.sync_copy(acc.at[pl.ds(s * RPT, RPT)], zbuf)
    pltpu.sync_copy(zbuf, outp_hbm.at[pl.ds(c * NP + s * RPT, RPT)])


EB = 1664


@functools.partial(
    pl.kernel,
    out_type=jax.ShapeDtypeStruct((NP, CH, CW), jnp.float32),
    mesh=_mesh,
    scratch_types=[
        [pltpu.VMEM((2, EB), jnp.int32) for _ in range(2)],
        [pltpu.VMEM((EB, CW), jnp.float32) for _ in range(2)],
        pltpu.VMEM_SHARED((NP, CW), jnp.float32),
        [pltpu.SemaphoreType.DMA for _ in range(2)],
        [pltpu.SemaphoreType.DMA for _ in range(2)],
    ],
    compiler_params=pltpu.CompilerParams(use_tc_tiling_on_sc=False),
)
def _seg_sc(h0, h1, h2, h3, h4, h5, h6, h7, h8, h9, h10, h11, h12, h13,
            h14, h15, sd_hbm, zeros_hbm, t_hbm, sd, gbuf, acc, gsem, ssem):
    c = lax.axis_index("c")
    s = lax.axis_index("s")
    tables = [h0, h1, h2, h3, h4, h5, h6, h7,
              h8, h9, h10, h11, h12, h13, h14, h15]
    def _run(chunk, table, mine):
        e0 = s * 53248

        def _idx_gather(b, j):
            pltpu.sync_copy(sd_hbm.at[:, pl.ds(e0 + b * EB, EB)], sd[j])
            pltpu.async_copy(table.at[sd[j].at[0]], gbuf[j], gsem[j])

        def _wait_scatter(j):
            pltpu.make_async_copy(gbuf[j], acc.at[sd[j].at[1]],
                                  ssem[j]).wait()

        def _scatter(j):
            pltpu.make_async_copy(table.at[sd[j].at[0]], gbuf[j],
                                  gsem[j]).wait()
            pltpu.async_copy(gbuf[j], acc.at[sd[j].at[1]], ssem[j], add=True)

        @pl.when(mine)
        def _zero():
            pltpu.sync_copy(zeros_hbm, gbuf[0].at[pl.ds(0, 784)])
            for z in range(4):
                pltpu.sync_copy(gbuf[0].at[pl.ds(0, 784)],
                                acc.at[pl.ds(s * RPT + z * 784, 784)])

        plsc.subcore_barrier()

        @pl.when(mine)
        def _edges():
            def grp(i, carry):
                @pl.when(i > 0)
                def _drain():
                    for j in range(2):
                        _wait_scatter(j)

                for j in range(2):
                    _idx_gather(i * 2 + j, j)
                for j in range(2):
                    _scatter(j)
                return carry

            lax.fori_loop(0, 16, grp, 0)
            for j in range(2):
                _wait_scatter(j)

        plsc.subcore_barrier()

        @pl.when(mine)
        def _out():
            for z in range(4):
                pltpu.sync_copy(acc.at[pl.ds(s * RPT + z * 784, 784)],
                                gbuf[0].at[pl.ds(0, 784)])
                pltpu.sync_copy(gbuf[0].at[pl.ds(0, 784)],
                                t_hbm.at[pl.ds(s * RPT + z * 784, 784), chunk])

        plsc.subcore_barrier()

    for chunk in range(CH):
        _run(chunk, tables[chunk], c == (chunk // (CH // 2)))


def _prep_body(x_ref, da_ref, db_ref, dinv_ref, xs_ref):
    deg = da_ref[...] + db_ref[...]
    dinv = lax.rsqrt(jnp.maximum(deg, 1e-12))
    dinv_ref[...] = dinv[:, None]
    xs_ref[...] = dinv * x_ref[:, 0]


def _prep_tc(xpad, degp):
    return pl.pallas_call(
        _prep_body,
        grid=(BLK,),
        in_specs=[
            pl.BlockSpec((TB, 1), lambda i: (i, 0)),
            pl.BlockSpec((TB,), lambda i: (i,)),
            pl.BlockSpec((TB,), lambda i: (i + BLK,)),
        ],
        out_specs=[
            pl.BlockSpec((TB, 1), lambda i: (i, 0)),
            pl.BlockSpec((TB,), lambda i: (i,)),
        ],
        out_shape=[
            jax.ShapeDtypeStruct((NP, 1), jnp.float32),
            jax.ShapeDtypeStruct((NP,), jnp.float32),
        ],
    )(xpad, degp, degp)


def _l1_body(ta_ref, tb_ref, dinv_ref, w1_ref, b1_ref, *out_refs):
    t1 = ta_ref[...] + tb_ref[...]
    dinv = dinv_ref[:, 0]
    u = dinv * t1
    y = jax.nn.relu(u[:, None] * w1_ref[0, :][None, :] + b1_ref[...])
    hs = dinv[:, None] * y
    for cc in range(CH):
        out_refs[cc][...] = hs[:, cc * CW:(cc + 1) * CW]


def _l1_tc(t1p, dinv, W1, b1r):
    return pl.pallas_call(
        _l1_body,
        grid=(BLK,),
        in_specs=[
            pl.BlockSpec((TB,), lambda i: (i,)),
            pl.BlockSpec((TB,), lambda i: (i + BLK,)),
            pl.BlockSpec((TB, 1), lambda i: (i, 0)),
            pl.BlockSpec((1, H), lambda i: (0, 0)),
            pl.BlockSpec((1, H), lambda i: (0, 0)),
        ],
        out_specs=[pl.BlockSpec((TB, CW), lambda i: (i, 0))
                   for _ in range(CH)],
        out_shape=[jax.ShapeDtypeStruct((NP, CW), jnp.float32)
                   for _ in range(CH)],
    )(t1p, t1p, dinv, W1, b1r)


def _mid_body(t_ref, dinv_ref, w_ref, b_ref, *out_refs):
    dinv = dinv_ref[...]
    u = dinv * t_ref[...].reshape(TB, H)
    y = jax.nn.relu(
        jnp.dot(u, w_ref[...], preferred_element_type=jnp.float32)
        + b_ref[...])
    hs = dinv * y
    for cc in range(CH):
        out_refs[cc][...] = hs[:, cc * CW:(cc + 1) * CW]


def _mid_tc(t, dinv, W, br):
    return pl.pallas_call(
        _mid_body,
        grid=(BLK,),
        in_specs=[
            pl.BlockSpec((TB * H,), lambda i: (i,)),
            pl.BlockSpec((TB, 1), lambda i: (i, 0)),
            pl.BlockSpec((H, H), lambda i: (0, 0)),
            pl.BlockSpec((1, H), lambda i: (0, 0)),
        ],
        out_specs=[pl.BlockSpec((TB, CW), lambda i: (i, 0))
                   for _ in range(CH)],
        out_shape=[jax.ShapeDtypeStruct((NP, CW), jnp.float32)
                   for _ in range(CH)],
    )(t, dinv, W, br)


def _end_body(t_ref, dinv_ref, w_ref, b_ref, wc_ref, bc_ref, out_ref):
    dinv = dinv_ref[...]
    u = dinv * t_ref[...].reshape(TB, H)
    y = jax.nn.relu(
        jnp.dot(u, w_ref[...], preferred_element_type=jnp.float32)
        + b_ref[...])
    out_ref[...] = (
        jnp.dot(y, wc_ref[...], preferred_element_type=jnp.float32)
        + bc_ref[...])


def _end_tc(t, dinv, W, br, Wc, bcr):
    return pl.pallas_call(
        _end_body,
        grid=(BLK,),
        in_specs=[
            pl.BlockSpec((TB * H,), lambda i: (i,)),
            pl.BlockSpec((TB, 1), lambda i: (i, 0)),
            pl.BlockSpec((H, H), lambda i: (0, 0)),
            pl.BlockSpec((1, H), lambda i: (0, 0)),
            pl.BlockSpec((H, 1), lambda i: (0, 0)),
            pl.BlockSpec((1, 1), lambda i: (0, 0)),
        ],
        out_specs=pl.BlockSpec((TB, 1), lambda i: (i, 0)),
        out_shape=jax.ShapeDtypeStruct((NP, 1), jnp.float32),
    )(t, dinv, W, br, Wc, bcr)


def kernel(x, edge_index, W1, b1, W, b, Wc, bc):
    loops = jnp.arange(N, dtype=jnp.int32)
    pad_e = E2P - E2
    src2 = jnp.concatenate(
        [edge_index[0], loops, jnp.zeros((pad_e,), jnp.int32)])
    dst2 = jnp.concatenate(
        [edge_index[1], loops, jnp.full((pad_e,), NP - 1, jnp.int32)])
    dstr = dst2

    sd2 = jnp.stack([src2, dst2])
    xpad = jnp.zeros((NP, 1), jnp.float32).at[:N].set(x)
    ones2d = jnp.ones((1024,), jnp.float32)
    zeros1 = jnp.zeros((RPT,), jnp.float32)
    zeros32 = jnp.zeros((784, CW), jnp.float32)
    b1r = b1.reshape(1, H)
    br = b.reshape(1, H)
    bcr = bc.reshape(1, 1)

    degp = _deg_sc(dstr, ones2d, zeros1)
    dinv, xs = _prep_tc(xpad, degp)
    t1p = _seg1_sc(xs, src2, dstr, zeros1)
    hsf = _l1_tc(t1p, dinv, W1, b1r)
    for _ in range(2, LAYERS):
        t = _seg_sc(*hsf, sd2, zeros32)
        hsf = _mid_tc(t.reshape(NP * H), dinv, W, br)
    t = _seg_sc(*hsf, sd2, zeros32)
    out = _end_tc(t.reshape(NP * H), dinv, W, br, Wc, bcr)
    return out[:N]
